# baseline (speedup 1.0000x reference)
import functools

import jax
import jax.numpy as jnp
from jax import lax
from jax.experimental import pallas as pl
from jax.experimental.pallas import tpu as pltpu

_TOP_K = 8
_NEG = -1000000000.0


def _qkv_body(x_ref, w_ref, b_ref, q_ref, k_ref, vext_ref, *, nt):
    n = pl.program_id(1)
    y = jnp.dot(x_ref[0], w_ref[...], preferred_element_type=jnp.float32)
    y = y + b_ref[...]
    d = vext_ref.shape[-1]

    @pl.when(n < nt)
    def _():
        q_ref[...] = y[:, :d][None]
        k_ref[...] = y[:, d : 2 * d][None]

    vext_ref[...] = y[:, 2 * d :].astype(jnp.bfloat16)[None]


def _mj_from_tri(s):
    mf = jnp.floor((jnp.sqrt(8.0 * s.astype(jnp.float32) + 1.0) - 1.0) * 0.5)
    m = mf.astype(jnp.int32)
    m = jnp.where((m + 1) * (m + 2) // 2 <= s, m + 1, m)
    m = jnp.where(m * (m + 1) // 2 > s, m - 1, m)
    return m, s - m * (m + 1) // 2


def _score_body(qm_ref, kb_ref, out_ref, *, c):
    m, j = _mj_from_tri(pl.program_id(1))

    @pl.when(pl.program_id(1) == 0)
    def _():
        out_ref[...] = jnp.zeros_like(out_ref)

    rows = lax.broadcasted_iota(jnp.int32, (c, c), 0)
    cols = lax.broadcasted_iota(jnp.int32, (c, c), 1)
    low_mask = cols <= rows
    dn = (((1,), (1,)), ((), ()))

    if True:
        h = lax.dot_general(kb_ref[0], qm_ref[0], dn,
                            preferred_element_type=jnp.float32)
        r = pltpu.roll(h, 1, 1, stride=1, stride_axis=0)
        s_all = jnp.sum(r, axis=0)
        s_low = jnp.sum(jnp.where(low_mask, r, 0.0), axis=0)
        row = m - j
        off = pl.multiple_of(row * c, c)
        out_ref[0, 0, pl.ds(off, c)] += s_low

        @pl.when(row >= 1)
        def _():
            off2 = pl.multiple_of((row - 1) * c, c)
            out_ref[0, 0, pl.ds(off2, c)] += s_all - s_low


def _topk_body(score_ref, lags_ref, *, kk):
    b, t = score_ref.shape
    lane = lax.broadcasted_iota(jnp.int32, (b, t), 1)
    s = jnp.where(lane == 0, _NEG, score_ref[...])
    lags_ref[...] = jnp.zeros_like(lags_ref)
    for step in range(kk):
        m = jnp.max(s, axis=1, keepdims=True)
        idx = jnp.min(jnp.where(s == m, lane, t), axis=1, keepdims=True)
        lags_ref[:, step : step + 1] = idx
        s = jnp.where(lane == idx, _NEG, s)


def _shift_body(lags_ref, vext_ref, out_ref, *, cs, t, kk):
    n = pl.program_id(0)
    r0 = n * cs
    sel_r = lax.broadcasted_iota(jnp.int32, (cs, cs + 8), 0)
    sel_c = lax.broadcasted_iota(jnp.int32, (cs, cs + 8), 1)
    acc = None
    for i in range(kk):
        lag = lags_ref[i]
        start = lax.rem(r0 - lag + t, t)
        base = pl.multiple_of((start // 8) * 8, 8)
        rem8 = start - base
        blk = vext_ref[pl.ds(base, cs + 8), :]
        sel = jnp.where(sel_c == sel_r + rem8, 1.0, 0.0).astype(jnp.bfloat16)
        rows = jnp.dot(sel, blk, preferred_element_type=jnp.float32)
        acc = rows if acc is None else acc + rows
    out_ref[...] = acc * (1.0 / kk)


def kernel(x, Wq, bq, Wk, bk, Wv, bv):
    B, T, D = x.shape
    C = 512 if T % 512 == 0 else max(
        c for c in (256, 128, 64, 32, 16, 8) if T % c == 0)
    C = min(C, T)
    N = T // C
    kk = min(_TOP_K, T - 1)

    w = jnp.concatenate([Wq.T, Wk.T, Wv.T], axis=1)
    bias = jnp.concatenate([bq, bk, bv]).reshape(1, 3 * D)

    q, k, vext = pl.pallas_call(
        functools.partial(_qkv_body, nt=N),
        grid=(B, N + 1),
        in_specs=[
            pl.BlockSpec((1, C, D),
                         lambda b, n: (b, jnp.where(n == N, 0, n), 0)),
            pl.BlockSpec((D, 3 * D), lambda b, n: (0, 0)),
            pl.BlockSpec((1, 3 * D), lambda b, n: (0, 0)),
        ],
        out_specs=[
            pl.BlockSpec((1, C, D),
                         lambda b, n: (b, jnp.minimum(n, N - 1), 0)),
            pl.BlockSpec((1, C, D),
                         lambda b, n: (b, jnp.minimum(n, N - 1), 0)),
            pl.BlockSpec((1, C, D), lambda b, n: (b, n, 0)),
        ],
        out_shape=[
            jax.ShapeDtypeStruct((B, T, D), jnp.float32),
            jax.ShapeDtypeStruct((B, T, D), jnp.float32),
            jax.ShapeDtypeStruct((B, T + C, D), jnp.bfloat16),
        ],
    )(x, w, bias)
    krev = jnp.flip(k, axis=1)

    NTRI = N * (N + 1) // 2

    def _q_idx(b, s):
        m, _ = _mj_from_tri(s)
        return (b, m, 0)

    def _k_idx(b, s):
        m, j = _mj_from_tri(s)
        return (b, N - 1 - j, 0)

    score = pl.pallas_call(
        functools.partial(_score_body, c=C),
        grid=(B, NTRI),
        in_specs=[
            pl.BlockSpec((1, C, D), _q_idx),
            pl.BlockSpec((1, C, D), _k_idx),
        ],
        out_specs=pl.BlockSpec((1, 1, T), lambda b, s: (b, 0, 0)),
        out_shape=jax.ShapeDtypeStruct((B, 1, T), jnp.float32),
    )(q, krev)
    score = score.reshape(B, T)

    lags = pl.pallas_call(
        functools.partial(_topk_body, kk=kk),
        in_specs=[pl.BlockSpec((B, T), lambda: (0, 0))],
        out_specs=pl.BlockSpec((B, 128), lambda: (0, 0)),
        out_shape=jax.ShapeDtypeStruct((B, 128), jnp.int32),
    )(score)

    CS = min(64, C)
    shift_call = pl.pallas_call(
        functools.partial(_shift_body, cs=CS, t=T, kk=kk),
        grid=(T // CS,),
        in_specs=[
            pl.BlockSpec(memory_space=pltpu.SMEM),
            pl.BlockSpec((T + C, D), lambda n: (0, 0)),
        ],
        out_specs=pl.BlockSpec((CS, D), lambda n: (n, 0)),
        out_shape=jax.ShapeDtypeStruct((T, D), jnp.float32),
    )
    out = jnp.stack([shift_call(lags[b, :kk], vext[b]) for b in range(B)])
    return out


# speedup vs baseline: 1.2641x; 1.2641x over previous
import functools

import jax
import jax.numpy as jnp
from jax import lax
from jax.experimental import pallas as pl
from jax.experimental.pallas import tpu as pltpu

_TOP_K = 8
_NEG = -1000000000.0


def _qkv_body(x_ref, w_ref, b_ref, q_ref, k_ref, vext_ref, *, nt):
    n = pl.program_id(1)
    y = jnp.dot(x_ref[0], w_ref[...], preferred_element_type=jnp.float32)
    y = y + b_ref[...]
    d = vext_ref.shape[-1]

    @pl.when(n < nt)
    def _():
        q_ref[...] = y[:, :d][None]
        k_ref[...] = y[:, d : 2 * d][None]

    vext_ref[...] = y[:, 2 * d :].astype(jnp.bfloat16)[None]


def _mj_from_tri(s):
    mf = jnp.floor((jnp.sqrt(8.0 * s.astype(jnp.float32) + 1.0) - 1.0) * 0.5)
    m = mf.astype(jnp.int32)
    m = jnp.where((m + 1) * (m + 2) // 2 <= s, m + 1, m)
    m = jnp.where(m * (m + 1) // 2 > s, m - 1, m)
    return m, s - m * (m + 1) // 2


def _score_body(qm_ref, kb_ref, out_ref, *, c):
    m, j = _mj_from_tri(pl.program_id(1))

    @pl.when(pl.program_id(1) == 0)
    def _():
        out_ref[...] = jnp.zeros_like(out_ref)

    rows = lax.broadcasted_iota(jnp.int32, (c, c), 0)
    cols = lax.broadcasted_iota(jnp.int32, (c, c), 1)
    low_mask = cols <= rows
    dn = (((1,), (1,)), ((), ()))

    if True:
        h = lax.dot_general(kb_ref[0], qm_ref[0], dn,
                            preferred_element_type=jnp.float32)
        r = pltpu.roll(h, 1, 1, stride=1, stride_axis=0)
        s_all = jnp.sum(r, axis=0)
        s_low = jnp.sum(jnp.where(low_mask, r, 0.0), axis=0)
        row = m - j
        off = pl.multiple_of(row * c, c)
        out_ref[0, 0, pl.ds(off, c)] += s_low

        @pl.when(row >= 1)
        def _():
            off2 = pl.multiple_of((row - 1) * c, c)
            out_ref[0, 0, pl.ds(off2, c)] += s_all - s_low


def _topk_body(score_ref, lags_ref, *, kk):
    b, t = score_ref.shape
    lane = lax.broadcasted_iota(jnp.int32, (b, t), 1)
    s = jnp.where(lane == 0, _NEG, score_ref[...])
    lags_ref[...] = jnp.zeros_like(lags_ref)
    for step in range(kk):
        m = jnp.max(s, axis=1, keepdims=True)
        idx = jnp.min(jnp.where(s == m, lane, t), axis=1, keepdims=True)
        lags_ref[:, step : step + 1] = idx
        s = jnp.where(lane == idx, _NEG, s)


def _shift_body(lags_ref, vext_ref, out_ref, *, cs, t, kk):
    n = pl.program_id(0)
    r0 = n * cs
    sel_r = lax.broadcasted_iota(jnp.int32, (cs, cs + 8), 0)
    sel_c = lax.broadcasted_iota(jnp.int32, (cs, cs + 8), 1)
    acc = None
    for i in range(kk):
        lag = lags_ref[i]
        start = lax.rem(r0 - lag + t, t)
        base = pl.multiple_of((start // 8) * 8, 8)
        rem8 = start - base
        blk = vext_ref[pl.ds(base, cs + 8), :]
        sel = jnp.where(sel_c == sel_r + rem8, 1.0, 0.0).astype(jnp.bfloat16)
        rows = jnp.dot(sel, blk, preferred_element_type=jnp.float32)
        acc = rows if acc is None else acc + rows
    out_ref[...] = acc * (1.0 / kk)


def kernel(x, Wq, bq, Wk, bk, Wv, bv):
    B, T, D = x.shape
    C = 1024 if T % 1024 == 0 else max(
        c for c in (512, 256, 128, 64, 32, 16, 8) if T % c == 0)
    C = min(C, T)
    N = T // C
    kk = min(_TOP_K, T - 1)

    w = jnp.concatenate([Wq.T, Wk.T, Wv.T], axis=1)
    bias = jnp.concatenate([bq, bk, bv]).reshape(1, 3 * D)

    q, k, vext = pl.pallas_call(
        functools.partial(_qkv_body, nt=N),
        grid=(B, N + 1),
        in_specs=[
            pl.BlockSpec((1, C, D),
                         lambda b, n: (b, jnp.where(n == N, 0, n), 0)),
            pl.BlockSpec((D, 3 * D), lambda b, n: (0, 0)),
            pl.BlockSpec((1, 3 * D), lambda b, n: (0, 0)),
        ],
        out_specs=[
            pl.BlockSpec((1, C, D),
                         lambda b, n: (b, jnp.minimum(n, N - 1), 0)),
            pl.BlockSpec((1, C, D),
                         lambda b, n: (b, jnp.minimum(n, N - 1), 0)),
            pl.BlockSpec((1, C, D), lambda b, n: (b, n, 0)),
        ],
        out_shape=[
            jax.ShapeDtypeStruct((B, T, D), jnp.float32),
            jax.ShapeDtypeStruct((B, T, D), jnp.float32),
            jax.ShapeDtypeStruct((B, T + C, D), jnp.bfloat16),
        ],
    )(x, w, bias)
    krev = jnp.flip(k, axis=1)

    NTRI = N * (N + 1) // 2

    def _q_idx(b, s):
        m, _ = _mj_from_tri(s)
        return (b, m, 0)

    def _k_idx(b, s):
        m, j = _mj_from_tri(s)
        return (b, N - 1 - j, 0)

    score = pl.pallas_call(
        functools.partial(_score_body, c=C),
        grid=(B, NTRI),
        in_specs=[
            pl.BlockSpec((1, C, D), _q_idx),
            pl.BlockSpec((1, C, D), _k_idx),
        ],
        out_specs=pl.BlockSpec((1, 1, T), lambda b, s: (b, 0, 0)),
        out_shape=jax.ShapeDtypeStruct((B, 1, T), jnp.float32),
    )(q, krev)
    score = score.reshape(B, T)

    lags = pl.pallas_call(
        functools.partial(_topk_body, kk=kk),
        in_specs=[pl.BlockSpec((B, T), lambda: (0, 0))],
        out_specs=pl.BlockSpec((B, 128), lambda: (0, 0)),
        out_shape=jax.ShapeDtypeStruct((B, 128), jnp.int32),
    )(score)

    CS = min(128, C)
    shift_call = pl.pallas_call(
        functools.partial(_shift_body, cs=CS, t=T, kk=kk),
        grid=(T // CS,),
        in_specs=[
            pl.BlockSpec(memory_space=pltpu.SMEM),
            pl.BlockSpec((T + C, D), lambda n: (0, 0)),
        ],
        out_specs=pl.BlockSpec((CS, D), lambda n: (n, 0)),
        out_shape=jax.ShapeDtypeStruct((T, D), jnp.float32),
    )
    out = jnp.stack([shift_call(lags[b, :kk], vext[b]) for b in range(B)])
    return out
